# Initial kernel scaffold; baseline (speedup 1.0000x reference)
import jax
import jax.numpy as jnp
from jax.experimental import pallas as pl


def kernel(h_s, h_q, support_labels, query_labels, n_way, W1, b1, g1, be1, W2, b2, g2, be2, Wc, bc):
    raise NotImplementedError("write your pallas kernel here")


# speedup vs baseline: 1.6626x; 1.6626x over previous
import functools
import jax
import jax.numpy as jnp
import numpy as np
from jax.experimental import pallas as pl
from jax.experimental.pallas import tpu as pltpu
from jax.experimental.pallas import tpu_sc as plsc

KK = 32
DD = 128
MP = 10240
MREAL = 10000
CHUNK = 16
NCH = MP // CHUNK
RB = 256
CB = 2048
NEG = np.float32(-np.inf)


def _aff_body(h_ref, hc_ref, aff_ref, cm_ref):
    i = pl.program_id(0)
    j = pl.program_id(1)
    a = h_ref[...]
    b = hc_ref[...]
    an = a * jax.lax.rsqrt(jnp.maximum(jnp.sum(a * a, 1, keepdims=True), 1e-24))
    bn = b * jax.lax.rsqrt(jnp.maximum(jnp.sum(b * b, 1, keepdims=True), 1e-24))
    t = jax.lax.dot_general(an, bn, (((1,), (1,)), ((), ())),
                            preferred_element_type=jnp.float32,
                            precision=jax.lax.Precision.DEFAULT)
    grow = i * RB + jax.lax.broadcasted_iota(jnp.int32, (RB, CB), 0)
    gcol = j * CB + jax.lax.broadcasted_iota(jnp.int32, (RB, CB), 1)
    t = jnp.where((grow == gcol) | (gcol >= MREAL), NEG, t)
    aff_ref[...] = t
    cm_ref[...] = jnp.max(t.reshape(RB, CB // CHUNK, CHUNK), axis=2)


def _p1_aff(hp):
    return pl.pallas_call(
        _aff_body,
        grid=(MP // RB, MP // CB),
        in_specs=[
            pl.BlockSpec((RB, DD), lambda i, j: (i, 0)),
            pl.BlockSpec((CB, DD), lambda i, j: (j, 0)),
        ],
        out_specs=[
            pl.BlockSpec((RB, CB), lambda i, j: (i, j)),
            pl.BlockSpec((RB, CB // CHUNK), lambda i, j: (i, j)),
        ],
        out_shape=[
            jax.ShapeDtypeStruct((MP, MP), jnp.float32),
            jax.ShapeDtypeStruct((MP, NCH), jnp.float32),
        ],
    )(hp, hp)


def _tlo_body(cm_ref, tlo_ref):
    x = cm_ref[...]
    m = jnp.max(x, axis=1, keepdims=True)
    for _ in range(KK - 1):
        x = jnp.where(x >= m, NEG, x)
        m = jnp.max(x, axis=1, keepdims=True)
    tlo_ref[...] = jnp.broadcast_to(m, tlo_ref.shape)


def _p2_tlo(cm):
    return pl.pallas_call(
        _tlo_body,
        grid=(MP // RB,),
        in_specs=[pl.BlockSpec((RB, NCH), lambda i: (i, 0))],
        out_specs=pl.BlockSpec((RB, 8), lambda i: (i, 0)),
        out_shape=jax.ShapeDtypeStruct((MP, 8), jnp.float32),
    )(cm)


def _mm_body(a_ref, b_ref, o_ref):
    o_ref[...] = jax.lax.dot_general(
        a_ref[...], b_ref[...], (((1,), (0,)), ((), ())),
        preferred_element_type=jnp.float32,
        precision=jax.lax.Precision.DEFAULT)


def _tc_matmul(a, b):
    m, k = a.shape
    _, n = b.shape
    return pl.pallas_call(
        _mm_body,
        grid=(m // 512,),
        in_specs=[pl.BlockSpec((512, k), lambda i: (i, 0)),
                  pl.BlockSpec((k, n), lambda i: (0, 0))],
        out_specs=pl.BlockSpec((512, n), lambda i: (i, 0)),
        out_shape=jax.ShapeDtypeStruct((m, n), jnp.float32),
    )(a, b)


def _degu_body(degp_ref, s_ref, xw_ref, u_ref, dinv_ref):
    s = s_ref[...]
    deg = degp_ref[0, :] + degp_ref[1, :] + jnp.sum(s, axis=1) + 1.0
    dinv = jax.lax.rsqrt(deg)
    dinv_ref[...] = jnp.broadcast_to(dinv[:, None], dinv_ref.shape)
    u_ref[...] = xw_ref[...] * dinv[:, None]


def _tc_degu(degp, s2d, xw1):
    return pl.pallas_call(
        _degu_body,
        grid=(MP // 512,),
        in_specs=[pl.BlockSpec((2, 512), lambda i: (0, i)),
                  pl.BlockSpec((512, KK), lambda i: (i, 0)),
                  pl.BlockSpec((512, DD), lambda i: (i, 0))],
        out_specs=[pl.BlockSpec((512, DD), lambda i: (i, 0)),
                   pl.BlockSpec((512, 8), lambda i: (i, 0))],
        out_shape=[jax.ShapeDtypeStruct((MP, DD), jnp.float32),
                   jax.ShapeDtypeStruct((MP, 8), jnp.float32)],
    )(degp, s2d, xw1)


TB = 512


def _comb_body(fwd_ref, acc_ref, u_ref, dinv_ref, b_ref, out_ref, st_ref):
    i = pl.program_id(0)
    v = fwd_ref[...] + acc_ref[0] + acc_ref[1] + u_ref[...]
    out = v * dinv_ref[:, 0][:, None] + b_ref[0, :][None, :]
    rmask = (i * TB + jax.lax.broadcasted_iota(jnp.int32, out.shape, 0)) < MREAL
    mz = jnp.where(rmask, out, 0.0)
    out_ref[...] = out

    @pl.when(i == 0)
    def _():
        st_ref[...] = jnp.zeros_like(st_ref)
    st_ref[0, :] += jnp.sum(mz, axis=0)
    st_ref[1, :] += jnp.sum(mz * mz, axis=0)


def _tc_combine(fwd, acc, u, dinv, b):
    return pl.pallas_call(
        _comb_body,
        grid=(MP // TB,),
        in_specs=[pl.BlockSpec((TB, DD), lambda i: (i, 0)),
                  pl.BlockSpec((2, TB, DD), lambda i: (0, i, 0)),
                  pl.BlockSpec((TB, DD), lambda i: (i, 0)),
                  pl.BlockSpec((TB, 8), lambda i: (i, 0)),
                  pl.BlockSpec((1, DD), lambda i: (0, 0))],
        out_specs=[pl.BlockSpec((TB, DD), lambda i: (i, 0)),
                   pl.BlockSpec((2, DD), lambda i: (0, 0))],
        out_shape=[jax.ShapeDtypeStruct((MP, DD), jnp.float32),
                   jax.ShapeDtypeStruct((2, DD), jnp.float32)],
    )(fwd, acc, u, dinv, b.reshape(1, DD))


def _bn_z(out, st_ref, g_ref, be_ref):
    mean = st_ref[0, :] / MREAL
    var = st_ref[1, :] / MREAL - mean * mean
    return ((out - mean[None, :]) * jax.lax.rsqrt(var + 1e-5)[None, :]
            * g_ref[0, :][None, :] + be_ref[0, :][None, :])


def _apply1_body(out_ref, st_ref, g_ref, be_ref, w2_ref, dinv_ref, u2_ref):
    z = jnp.maximum(_bn_z(out_ref[...], st_ref, g_ref, be_ref), 0.0)
    xw2 = jax.lax.dot_general(z, w2_ref[...], (((1,), (0,)), ((), ())),
                              preferred_element_type=jnp.float32,
                              precision=jax.lax.Precision.DEFAULT)
    u2_ref[...] = xw2 * dinv_ref[:, 0][:, None]


def _tc_apply1(out, st, g, be, w2, dinv):
    return pl.pallas_call(
        _apply1_body,
        grid=(MP // TB,),
        in_specs=[pl.BlockSpec((TB, DD), lambda i: (i, 0)),
                  pl.BlockSpec((2, DD), lambda i: (0, 0)),
                  pl.BlockSpec((1, DD), lambda i: (0, 0)),
                  pl.BlockSpec((1, DD), lambda i: (0, 0)),
                  pl.BlockSpec((DD, DD), lambda i: (0, 0)),
                  pl.BlockSpec((TB, 8), lambda i: (i, 0))],
        out_specs=pl.BlockSpec((TB, DD), lambda i: (i, 0)),
        out_shape=jax.ShapeDtypeStruct((MP, DD), jnp.float32),
    )(out, st, g.reshape(1, DD), be.reshape(1, DD), w2, dinv)


def _apply2_body(out_ref, st_ref, g_ref, be_ref, wc_ref, bc_ref, ql_ref,
                 cmsk_ref, lg_ref, loss_ref):
    i = pl.program_id(0)
    z = _bn_z(out_ref[...], st_ref, g_ref, be_ref)
    lg = jax.lax.dot_general(z, wc_ref[...], (((1,), (0,)), ((), ())),
                             preferred_element_type=jnp.float32,
                             precision=jax.lax.Precision.DEFAULT)
    lg = lg + bc_ref[0, :][None, :]
    col = jax.lax.broadcasted_iota(jnp.int32, lg.shape, 1)
    colmask = cmsk_ref[0, :][None, :] > 0
    lg = jnp.where(colmask, lg, 0.0)
    lgm = jnp.where(colmask, lg, NEG)
    mx = jnp.max(lgm, axis=1, keepdims=True)
    lse = mx + jnp.log(jnp.sum(jnp.exp(lgm - mx), axis=1, keepdims=True))
    logp = lgm - lse
    qrow = i * TB + jax.lax.broadcasted_iota(jnp.int32, lg.shape, 0)
    qmask = (qrow >= MREAL // 2) & (qrow < MREAL)
    pick = jnp.where(qmask & colmask & (col == ql_ref[:, 0][:, None]),
                     logp, 0.0)
    lg_ref[...] = lg

    @pl.when(i == 0)
    def _():
        loss_ref[0, 0] = 0.0
    loss_ref[0, 0] += -jnp.sum(pick) / (MREAL // 2)


def _tc_apply2(out, st, g, be, wcp, bcp, qlp, cmsk):
    return pl.pallas_call(
        _apply2_body,
        grid=(MP // TB,),
        in_specs=[pl.BlockSpec((TB, DD), lambda i: (i, 0)),
                  pl.BlockSpec((2, DD), lambda i: (0, 0)),
                  pl.BlockSpec((1, DD), lambda i: (0, 0)),
                  pl.BlockSpec((1, DD), lambda i: (0, 0)),
                  pl.BlockSpec((DD, DD), lambda i: (0, 0)),
                  pl.BlockSpec((1, DD), lambda i: (0, 0)),
                  pl.BlockSpec((TB, 1), lambda i: (i, 0)),
                  pl.BlockSpec((1, DD), lambda i: (0, 0))],
        out_specs=[pl.BlockSpec((TB, DD), lambda i: (i, 0)),
                   pl.BlockSpec((1, 1), lambda i: (0, 0),
                                memory_space=pltpu.SMEM)],
        out_shape=[jax.ShapeDtypeStruct((MP, DD), jnp.float32),
                   jax.ShapeDtypeStruct((1, 1), jnp.float32)],
    )(out, st, g.reshape(1, DD), be.reshape(1, DD), wcp, bcp, qlp, cmsk)


def _p3_topk_xla(aff, cm, tlo):
    vals, idx = jax.lax.top_k(aff[:MREAL, :MREAL], KK)
    s = jax.nn.sigmoid(vals)
    degp = jnp.zeros((2, MP), jnp.float32)
    degp = degp.at[0, idx.reshape(-1)].add(s.reshape(-1))
    idxp = jnp.zeros((MP, KK), jnp.int32)
    sp = jnp.zeros((MP, KK), jnp.float32)
    return (idxp.at[:MREAL].set(idx.astype(jnp.int32)),
            sp.at[:MREAL].set(s), degp)


def _p4_agg_xla(u, idx2d, s2d):
    fwd = jnp.einsum('rk,rkd->rd', s2d[:MREAL], u[idx2d[:MREAL]])
    fwd = jnp.zeros((MP, DD), jnp.float32).at[:MREAL].set(fwd)
    acc = jnp.zeros((MP, DD), jnp.float32)
    msg = s2d[:MREAL][:, :, None] * u[:MREAL][:, None, :]
    acc = acc.at[idx2d[:MREAL].reshape(-1)].add(msg.reshape(-1, DD))
    return fwd, jnp.stack([acc, jnp.zeros_like(acc)], 0)


def kernel(h_s, h_q, support_labels, query_labels, n_way,
           W1, b1, g1, be1, W2, b2, g2, be2, Wc, bc):
    h_all = jnp.concatenate([h_s, h_q], 0)
    hp = jnp.zeros((MP, DD), jnp.float32).at[:MREAL].set(h_all)

    aff, cm = _p1_aff(hp)
    tlo = _p2_tlo(cm)

    d_in = DD + Wc.shape[1]
    label_s = jax.nn.one_hot(support_labels, Wc.shape[1], dtype=jnp.float32)
    z = jnp.concatenate([h_all,
                         jnp.concatenate([label_s,
                                          jnp.zeros((h_q.shape[0], Wc.shape[1]),
                                                    jnp.float32)], 0)], 1)
    zp = jnp.zeros((MP, 256), jnp.float32).at[:MREAL, :d_in].set(z)
    w1p = jnp.zeros((256, DD), jnp.float32).at[:d_in].set(W1)
    xw1 = _tc_matmul(zp, w1p)

    idx2d, s2d, degp = _p3_topk_xla(aff, cm, tlo[:, 0])

    u1, dinv = _tc_degu(degp, s2d, xw1)
    fwd1, acc1 = _p4_agg_xla(u1, idx2d, s2d)
    out1, st1 = _tc_combine(fwd1, acc1, u1, dinv, b1)
    u2 = _tc_apply1(out1, st1, g1, be1, W2, dinv)
    fwd2, acc2 = _p4_agg_xla(u2, idx2d, s2d)
    out2, st2 = _tc_combine(fwd2, acc2, u2, dinv, b2)

    wcp = jnp.zeros((DD, DD), jnp.float32).at[:, :Wc.shape[1]].set(Wc)
    bcp = jnp.zeros((1, DD), jnp.float32).at[0, :Wc.shape[1]].set(bc)
    qlp = jnp.zeros((MP, 1), jnp.int32).at[MREAL // 2:MREAL, 0].set(query_labels)
    cmsk = ((jnp.arange(DD) < n_way) & (jnp.arange(DD) < Wc.shape[1])
            ).astype(jnp.float32).reshape(1, DD)
    lgp, loss = _tc_apply2(out2, st2, g2, be2, wcp, bcp, qlp, cmsk)
    return lgp[MREAL // 2:MREAL, :Wc.shape[1]], loss[0, 0]


# speedup vs baseline: 6.0586x; 3.6442x over previous
import functools
import jax
import jax.numpy as jnp
import numpy as np
from jax.experimental import pallas as pl
from jax.experimental.pallas import tpu as pltpu
from jax.experimental.pallas import tpu_sc as plsc

KK = 32
DD = 128
MP = 10240
MREAL = 10000
CHUNK = 16
NCH = MP // CHUNK
RB = 256
CB = 2048
NEG = np.float32(-np.inf)


def _aff_body(h_ref, hc_ref, aff_ref, cm_ref):
    i = pl.program_id(0)
    j = pl.program_id(1)
    a = h_ref[...]
    b = hc_ref[...]
    an = a * jax.lax.rsqrt(jnp.maximum(jnp.sum(a * a, 1, keepdims=True), 1e-24))
    bn = b * jax.lax.rsqrt(jnp.maximum(jnp.sum(b * b, 1, keepdims=True), 1e-24))
    t = jax.lax.dot_general(an, bn, (((1,), (1,)), ((), ())),
                            preferred_element_type=jnp.float32,
                            precision=jax.lax.Precision.DEFAULT)
    grow = i * RB + jax.lax.broadcasted_iota(jnp.int32, (RB, CB), 0)
    gcol = j * CB + jax.lax.broadcasted_iota(jnp.int32, (RB, CB), 1)
    t = jnp.where((grow == gcol) | (gcol >= MREAL), NEG, t)
    aff_ref[...] = t
    cm_ref[...] = jnp.max(t.reshape(RB, CB // CHUNK, CHUNK), axis=2)


def _p1_aff(hp):
    return pl.pallas_call(
        _aff_body,
        grid=(MP // RB, MP // CB),
        in_specs=[
            pl.BlockSpec((RB, DD), lambda i, j: (i, 0)),
            pl.BlockSpec((CB, DD), lambda i, j: (j, 0)),
        ],
        out_specs=[
            pl.BlockSpec((RB, CB), lambda i, j: (i, j)),
            pl.BlockSpec((RB, CB // CHUNK), lambda i, j: (i, j)),
        ],
        out_shape=[
            jax.ShapeDtypeStruct((MP, MP), jnp.float32),
            jax.ShapeDtypeStruct((MP, NCH), jnp.float32),
        ],
    )(hp, hp)


def _tlo_body(cm_ref, tlo_ref):
    x = cm_ref[...]
    m = jnp.max(x, axis=1, keepdims=True)
    for _ in range(KK - 1):
        x = jnp.where(x >= m, NEG, x)
        m = jnp.max(x, axis=1, keepdims=True)
    tlo_ref[...] = jnp.broadcast_to(m, tlo_ref.shape)


def _p2_tlo(cm):
    return pl.pallas_call(
        _tlo_body,
        grid=(MP // RB,),
        in_specs=[pl.BlockSpec((RB, NCH), lambda i: (i, 0))],
        out_specs=pl.BlockSpec((RB, 8), lambda i: (i, 0)),
        out_shape=jax.ShapeDtypeStruct((MP, 8), jnp.float32),
    )(cm)


def _mm_body(a_ref, b_ref, o_ref):
    o_ref[...] = jax.lax.dot_general(
        a_ref[...], b_ref[...], (((1,), (0,)), ((), ())),
        preferred_element_type=jnp.float32,
        precision=jax.lax.Precision.DEFAULT)


def _tc_matmul(a, b):
    m, k = a.shape
    _, n = b.shape
    return pl.pallas_call(
        _mm_body,
        grid=(m // 512,),
        in_specs=[pl.BlockSpec((512, k), lambda i: (i, 0)),
                  pl.BlockSpec((k, n), lambda i: (0, 0))],
        out_specs=pl.BlockSpec((512, n), lambda i: (i, 0)),
        out_shape=jax.ShapeDtypeStruct((m, n), jnp.float32),
    )(a, b)


def _degu_body(degp_ref, s_ref, xw_ref, u_ref, dinv_ref):
    s = s_ref[...]
    deg = jnp.sum(degp_ref[...], axis=0) + jnp.sum(s, axis=1) + 1.0
    dinv = jax.lax.rsqrt(deg)
    dinv_ref[...] = jnp.broadcast_to(dinv[:, None], dinv_ref.shape)
    u_ref[...] = xw_ref[...] * dinv[:, None]


def _tc_degu(degp, s2d, xw1):
    return pl.pallas_call(
        _degu_body,
        grid=(MP // 512,),
        in_specs=[pl.BlockSpec((32, 512), lambda i: (0, i)),
                  pl.BlockSpec((512, KK), lambda i: (i, 0)),
                  pl.BlockSpec((512, DD), lambda i: (i, 0))],
        out_specs=[pl.BlockSpec((512, DD), lambda i: (i, 0)),
                   pl.BlockSpec((512, 8), lambda i: (i, 0))],
        out_shape=[jax.ShapeDtypeStruct((MP, DD), jnp.float32),
                   jax.ShapeDtypeStruct((MP, 8), jnp.float32)],
    )(degp, s2d, xw1)


TB = 512


def _comb_body(fwd_ref, acc_ref, u_ref, dinv_ref, b_ref, out_ref, st_ref):
    i = pl.program_id(0)
    v = fwd_ref[...] + acc_ref[0] + acc_ref[1] + u_ref[...]
    out = v * dinv_ref[:, 0][:, None] + b_ref[0, :][None, :]
    rmask = (i * TB + jax.lax.broadcasted_iota(jnp.int32, out.shape, 0)) < MREAL
    mz = jnp.where(rmask, out, 0.0)
    out_ref[...] = out

    @pl.when(i == 0)
    def _():
        st_ref[...] = jnp.zeros_like(st_ref)
    st_ref[0, :] += jnp.sum(mz, axis=0)
    st_ref[1, :] += jnp.sum(mz * mz, axis=0)


def _tc_combine(fwd, acc, u, dinv, b):
    return pl.pallas_call(
        _comb_body,
        grid=(MP // TB,),
        in_specs=[pl.BlockSpec((TB, DD), lambda i: (i, 0)),
                  pl.BlockSpec((2, TB, DD), lambda i: (0, i, 0)),
                  pl.BlockSpec((TB, DD), lambda i: (i, 0)),
                  pl.BlockSpec((TB, 8), lambda i: (i, 0)),
                  pl.BlockSpec((1, DD), lambda i: (0, 0))],
        out_specs=[pl.BlockSpec((TB, DD), lambda i: (i, 0)),
                   pl.BlockSpec((2, DD), lambda i: (0, 0))],
        out_shape=[jax.ShapeDtypeStruct((MP, DD), jnp.float32),
                   jax.ShapeDtypeStruct((2, DD), jnp.float32)],
    )(fwd, acc, u, dinv, b.reshape(1, DD))


def _bn_z(out, st_ref, g_ref, be_ref):
    mean = st_ref[0, :] / MREAL
    var = st_ref[1, :] / MREAL - mean * mean
    return ((out - mean[None, :]) * jax.lax.rsqrt(var + 1e-5)[None, :]
            * g_ref[0, :][None, :] + be_ref[0, :][None, :])


def _apply1_body(out_ref, st_ref, g_ref, be_ref, w2_ref, dinv_ref, u2_ref):
    z = jnp.maximum(_bn_z(out_ref[...], st_ref, g_ref, be_ref), 0.0)
    xw2 = jax.lax.dot_general(z, w2_ref[...], (((1,), (0,)), ((), ())),
                              preferred_element_type=jnp.float32,
                              precision=jax.lax.Precision.DEFAULT)
    u2_ref[...] = xw2 * dinv_ref[:, 0][:, None]


def _tc_apply1(out, st, g, be, w2, dinv):
    return pl.pallas_call(
        _apply1_body,
        grid=(MP // TB,),
        in_specs=[pl.BlockSpec((TB, DD), lambda i: (i, 0)),
                  pl.BlockSpec((2, DD), lambda i: (0, 0)),
                  pl.BlockSpec((1, DD), lambda i: (0, 0)),
                  pl.BlockSpec((1, DD), lambda i: (0, 0)),
                  pl.BlockSpec((DD, DD), lambda i: (0, 0)),
                  pl.BlockSpec((TB, 8), lambda i: (i, 0))],
        out_specs=pl.BlockSpec((TB, DD), lambda i: (i, 0)),
        out_shape=jax.ShapeDtypeStruct((MP, DD), jnp.float32),
    )(out, st, g.reshape(1, DD), be.reshape(1, DD), w2, dinv)


def _apply2_body(out_ref, st_ref, g_ref, be_ref, wc_ref, bc_ref, ql_ref,
                 cmsk_ref, lg_ref, loss_ref):
    i = pl.program_id(0)
    z = _bn_z(out_ref[...], st_ref, g_ref, be_ref)
    lg = jax.lax.dot_general(z, wc_ref[...], (((1,), (0,)), ((), ())),
                             preferred_element_type=jnp.float32,
                             precision=jax.lax.Precision.DEFAULT)
    lg = lg + bc_ref[0, :][None, :]
    col = jax.lax.broadcasted_iota(jnp.int32, lg.shape, 1)
    colmask = cmsk_ref[0, :][None, :] > 0
    lg = jnp.where(colmask, lg, 0.0)
    lgm = jnp.where(colmask, lg, NEG)
    mx = jnp.max(lgm, axis=1, keepdims=True)
    lse = mx + jnp.log(jnp.sum(jnp.exp(lgm - mx), axis=1, keepdims=True))
    logp = lgm - lse
    qrow = i * TB + jax.lax.broadcasted_iota(jnp.int32, lg.shape, 0)
    qmask = (qrow >= MREAL // 2) & (qrow < MREAL)
    pick = jnp.where(qmask & colmask & (col == ql_ref[:, 0][:, None]),
                     logp, 0.0)
    lg_ref[...] = lg

    @pl.when(i == 0)
    def _():
        loss_ref[0, 0] = 0.0
    loss_ref[0, 0] += -jnp.sum(pick) / (MREAL // 2)


def _tc_apply2(out, st, g, be, wcp, bcp, qlp, cmsk):
    return pl.pallas_call(
        _apply2_body,
        grid=(MP // TB,),
        in_specs=[pl.BlockSpec((TB, DD), lambda i: (i, 0)),
                  pl.BlockSpec((2, DD), lambda i: (0, 0)),
                  pl.BlockSpec((1, DD), lambda i: (0, 0)),
                  pl.BlockSpec((1, DD), lambda i: (0, 0)),
                  pl.BlockSpec((DD, DD), lambda i: (0, 0)),
                  pl.BlockSpec((1, DD), lambda i: (0, 0)),
                  pl.BlockSpec((TB, 1), lambda i: (i, 0)),
                  pl.BlockSpec((1, DD), lambda i: (0, 0))],
        out_specs=[pl.BlockSpec((TB, DD), lambda i: (i, 0)),
                   pl.BlockSpec((1, 1), lambda i: (0, 0),
                                memory_space=pltpu.SMEM)],
        out_shape=[jax.ShapeDtypeStruct((MP, DD), jnp.float32),
                   jax.ShapeDtypeStruct((1, 1), jnp.float32)],
    )(out, st, g.reshape(1, DD), be.reshape(1, DD), wcp, bcp, qlp, cmsk)


NW = 32
RPW = MP // NW
CAP_Q = 64
QHI = 48
CAP_C = 256
CHI = 208
BIGI = np.int32(2**30)
INF = np.float32(np.inf)


def _scal(v):
    return jax.lax.squeeze(jax.lax.slice(v, (0,), (1,)), (0,))


def _sload_f(ref, i, iota16):
    off = (i // 16) * 16
    v = ref[pl.ds(off, 16)]
    return jnp.sum(jnp.where(iota16 == (i - off), v, 0.0))


def _sload_i(ref, i, iota16):
    off = (i // 16) * 16
    v = ref[pl.ds(off, 16)]
    return jnp.sum(jnp.where(iota16 == (i - off), v, 0))


def _p3_body(aff_h, cm_h, tlo_h, idx_h, s_h, degp_h,
             cmrow, tlov, qg, gbuf, cval, cidx, selv, seli, outi, outs,
             degloc, sem):
    cid = jax.lax.axis_index("c")
    sid = jax.lax.axis_index("s")
    wid = cid * 16 + sid
    base = wid * RPW
    iota16 = jax.lax.iota(jnp.int32, 16)
    zf16 = jnp.zeros((16,), jnp.float32)
    zi16 = jnp.zeros((16,), jnp.int32)

    pltpu.sync_copy(tlo_h.at[pl.ds(base, RPW)], tlov)

    def _zero(i, _):
        degloc[pl.ds(i * 16, 16)] = zf16
        return 0
    jax.lax.fori_loop(0, MP // 16, _zero, 0)
    for i in range(CAP_Q // 16):
        qg[pl.ds(i * 16, 16)] = zi16

    def select32(nc):
        def sel_k(k, pst):
            pv, pi = pst
            nchk = (nc + 15) // 16

            def sbody(cc, bst):
                bv, bi = bst
                v = cval[pl.ds(cc * 16, 16)]
                ii = cidx[pl.ds(cc * 16, 16)]
                valid = (cc * 16 + iota16) < nc
                elig = valid & ((v < pv) | ((v == pv) & (ii > pi)))
                vm = jnp.where(elig, v, NEG)
                m = jnp.max(vm)
                im = jnp.min(jnp.where(elig & (v == m), ii, BIGI))
                better = (m > bv) | ((m == bv) & (im < bi))
                return (jnp.where(better, m, bv), jnp.where(better, im, bi))

            bv, bi = jax.lax.fori_loop(0, nchk, sbody, (NEG, BIGI))
            lane0 = iota16 == 0
            kk = jnp.full((16,), k, jnp.int32)
            plsc.store_scatter(selv, [kk], jnp.full((16,), bv), mask=lane0)
            plsc.store_scatter(seli, [kk], jnp.full((16,), bi), mask=lane0)
            return (bv, bi)

        jax.lax.fori_loop(0, KK, sel_k, (INF, -1 * jnp.ones((), jnp.int32)))

    def process_row(rl, r):
        pltpu.sync_copy(cm_h.at[r], cmrow)
        t0 = _sload_f(tlov, rl, iota16)

        def flush(nq, nc, t, r):
            pltpu.async_copy(aff_h.at[qg], gbuf, sem).wait()

            def fbody(e, st2):
                nc, t = st2
                g = gbuf[e]
                gid = _sload_i(qg, e, iota16)
                colbase = (gid - r * NCH) * 16
                msk = g >= t
                cnt = _scal(plsc.all_reduce_population_count(msk))

                def do_ins(nc):
                    plsc.store_compressed(cval.at[pl.ds(nc, 16)], g, mask=msk)
                    plsc.store_compressed(cidx.at[pl.ds(nc, 16)],
                                          colbase + iota16, mask=msk)
                    return nc + cnt
                nc = jax.lax.cond(cnt > 0, do_ins, lambda nc: nc, nc)

                def do_comp(t):
                    select32(nc)
                    for h in range(KK // 16):
                        cval[pl.ds(h * 16, 16)] = selv[pl.ds(h * 16, 16)]
                        cidx[pl.ds(h * 16, 16)] = seli[pl.ds(h * 16, 16)]
                    return selv[pl.ds(16, 16)][15]
                t = jax.lax.cond(nc >= CHI, do_comp, lambda t: t, t)
                nc = jnp.where(nc >= CHI, KK, nc)
                return (nc, t)

            nc, t = jax.lax.fori_loop(0, nq, fbody, (nc, t))
            return (jnp.zeros((), jnp.int32), nc, t)

        def scan_body(cb, st):
            nq, nc, t = st
            v = cmrow[pl.ds(cb * 16, 16)]
            msk = v >= t
            cnt = _scal(plsc.all_reduce_population_count(msk))

            def do_store(nq):
                ids = r * NCH + cb * 16 + iota16
                plsc.store_compressed(qg.at[pl.ds(nq, 16)], ids, mask=msk)
                return nq + cnt
            nq = jax.lax.cond(cnt > 0, do_store, lambda nq: nq, nq)
            return jax.lax.cond(nq >= QHI,
                                lambda st: flush(st[0], st[1], st[2], r),
                                lambda st: st, (nq, nc, t))

        z32 = jnp.zeros((), jnp.int32)
        nq, nc, t = jax.lax.fori_loop(0, NCH // 16, scan_body, (z32, z32, t0))
        nq, nc, t = jax.lax.cond(nq > 0,
                                 lambda st: flush(st[0], st[1], st[2], r),
                                 lambda st: st, (nq, nc, t))
        select32(nc)
        for h in range(KK // 16):
            sv = selv[pl.ds(h * 16, 16)]
            si = seli[pl.ds(h * 16, 16)]
            sg = 1.0 / (1.0 + jnp.exp(-sv))
            outs[pl.ds(rl * KK + h * 16, 16)] = sg
            outi[pl.ds(rl * KK + h * 16, 16)] = si
            plsc.addupdate_scatter(degloc, [si], sg)

    def row_body(rl, _):
        r = base + rl

        @pl.when(r < MREAL)
        def _():
            process_row(rl, r)
        return 0

    jax.lax.fori_loop(0, RPW, row_body, 0)

    pltpu.sync_copy(outi, idx_h.at[pl.ds(base * KK, RPW * KK)])
    pltpu.sync_copy(outs, s_h.at[pl.ds(base * KK, RPW * KK)])
    pltpu.sync_copy(degloc, degp_h.at[wid])


def _p3_topk_sc(aff, cm, tlo_flat):
    mesh = plsc.VectorSubcoreMesh(core_axis_name="c", subcore_axis_name="s")
    affv = aff.reshape(MP * NCH, CHUNK)
    fn = pl.kernel(
        _p3_body,
        out_type=[jax.ShapeDtypeStruct((MP * KK,), jnp.int32),
                  jax.ShapeDtypeStruct((MP * KK,), jnp.float32),
                  jax.ShapeDtypeStruct((NW, MP), jnp.float32)],
        mesh=mesh,
        scratch_types=[pltpu.VMEM((NCH,), jnp.float32),
                       pltpu.VMEM((RPW,), jnp.float32),
                       pltpu.VMEM((CAP_Q,), jnp.int32),
                       pltpu.VMEM((CAP_Q, CHUNK), jnp.float32),
                       pltpu.VMEM((CAP_C,), jnp.float32),
                       pltpu.VMEM((CAP_C,), jnp.int32),
                       pltpu.VMEM((KK,), jnp.float32),
                       pltpu.VMEM((KK,), jnp.int32),
                       pltpu.VMEM((RPW * KK,), jnp.int32),
                       pltpu.VMEM((RPW * KK,), jnp.float32),
                       pltpu.VMEM((MP,), jnp.float32),
                       pltpu.SemaphoreType.DMA],
        compiler_params=pltpu.CompilerParams(needs_layout_passes=False,
                                             use_tc_tiling_on_sc=False),
    )
    idx_f, s_f, degp = fn(affv, cm, tlo_flat)
    return idx_f.reshape(MP, KK), s_f.reshape(MP, KK), degp


def _p3_topk_xla(aff, cm, tlo):
    vals, idx = jax.lax.top_k(aff[:MREAL, :MREAL], KK)
    s = jax.nn.sigmoid(vals)
    degp = jnp.zeros((2, MP), jnp.float32)
    degp = degp.at[0, idx.reshape(-1)].add(s.reshape(-1))
    idxp = jnp.zeros((MP, KK), jnp.int32)
    sp = jnp.zeros((MP, KK), jnp.float32)
    return (idxp.at[:MREAL].set(idx.astype(jnp.int32)),
            sp.at[:MREAL].set(s), degp)


def _p4_agg_xla(u, idx2d, s2d):
    fwd = jnp.einsum('rk,rkd->rd', s2d[:MREAL], u[idx2d[:MREAL]])
    fwd = jnp.zeros((MP, DD), jnp.float32).at[:MREAL].set(fwd)
    acc = jnp.zeros((MP, DD), jnp.float32)
    msg = s2d[:MREAL][:, :, None] * u[:MREAL][:, None, :]
    acc = acc.at[idx2d[:MREAL].reshape(-1)].add(msg.reshape(-1, DD))
    return fwd, jnp.stack([acc, jnp.zeros_like(acc)], 0)


def kernel(h_s, h_q, support_labels, query_labels, n_way,
           W1, b1, g1, be1, W2, b2, g2, be2, Wc, bc):
    h_all = jnp.concatenate([h_s, h_q], 0)
    hp = jnp.zeros((MP, DD), jnp.float32).at[:MREAL].set(h_all)

    aff, cm = _p1_aff(hp)
    tlo = _p2_tlo(cm)

    d_in = DD + Wc.shape[1]
    label_s = jax.nn.one_hot(support_labels, Wc.shape[1], dtype=jnp.float32)
    z = jnp.concatenate([h_all,
                         jnp.concatenate([label_s,
                                          jnp.zeros((h_q.shape[0], Wc.shape[1]),
                                                    jnp.float32)], 0)], 1)
    zp = jnp.zeros((MP, 256), jnp.float32).at[:MREAL, :d_in].set(z)
    w1p = jnp.zeros((256, DD), jnp.float32).at[:d_in].set(W1)
    xw1 = _tc_matmul(zp, w1p)

    idx2d, s2d, degp = _p3_topk_sc(aff, cm, tlo[:, 0])

    u1, dinv = _tc_degu(degp, s2d, xw1)
    fwd1, acc1 = _p4_agg_xla(u1, idx2d, s2d)
    out1, st1 = _tc_combine(fwd1, acc1, u1, dinv, b1)
    u2 = _tc_apply1(out1, st1, g1, be1, W2, dinv)
    fwd2, acc2 = _p4_agg_xla(u2, idx2d, s2d)
    out2, st2 = _tc_combine(fwd2, acc2, u2, dinv, b2)

    wcp = jnp.zeros((DD, DD), jnp.float32).at[:, :Wc.shape[1]].set(Wc)
    bcp = jnp.zeros((1, DD), jnp.float32).at[0, :Wc.shape[1]].set(bc)
    qlp = jnp.zeros((MP, 1), jnp.int32).at[MREAL // 2:MREAL, 0].set(query_labels)
    cmsk = ((jnp.arange(DD) < n_way) & (jnp.arange(DD) < Wc.shape[1])
            ).astype(jnp.float32).reshape(1, DD)
    lgp, loss = _tc_apply2(out2, st2, g2, be2, wcp, bcp, qlp, cmsk)
    return lgp[MREAL // 2:MREAL, :Wc.shape[1]], loss[0, 0]
